# baseline (speedup 1.0000x reference)
import functools

import numpy as np
import jax
import jax.numpy as jnp
from jax.experimental import pallas as pl
from jax.experimental.pallas import tpu as pltpu

_NUM_CLASSES = 4
_CONF_T = 0.05
_MAX_PER_CLASS = 100
_MAX_DET = 100
_SIGMA = 0.05
_NEG = -3.0e38
_BIG = 2**30


def _nms_kernel(pred_ref, anch_ref, outf_ref, outc_ref,
                x1_ref, y1_ref, x2_ref, y2_ref, *, B, n_real, ROWS):
    C = _NUM_CLASSES
    T = _MAX_PER_CLASS
    BR = B * ROWS

    def ch(ref, k):
        return ref[k * BR:(k + 1) * BR, :]

    cxa = ch(anch_ref, 0)
    cya = ch(anch_ref, 1)
    wa = ch(anch_ref, 2)
    ha = ch(anch_ref, 3)

    x = ch(pred_ref, 0) * wa + cxa
    y = ch(pred_ref, 1) * ha + cya
    bw = jnp.exp(ch(pred_ref, 2)) * wa
    bh = jnp.exp(ch(pred_ref, 3)) * ha
    x1 = x - bw / 2.0
    y1 = y - bh / 2.0
    x2 = x + bw / 2.0
    y2 = y + bh / 2.0
    area = (x2 - x1) * (y2 - y1)

    x1_ref[...] = x1
    y1_ref[...] = y1
    x2_ref[...] = x2
    y2_ref[...] = y2

    s0 = jax.nn.sigmoid(ch(pred_ref, 4))
    s1 = jax.nn.sigmoid(ch(pred_ref, 5))
    s2 = jax.nn.sigmoid(ch(pred_ref, 6))
    s3 = jax.nn.sigmoid(ch(pred_ref, 7))
    mx = jnp.maximum(jnp.maximum(s0, s1), jnp.maximum(s2, s3))

    rowi = jax.lax.broadcasted_iota(jnp.int32, (BR, 128), 0)
    lanei = jax.lax.broadcasted_iota(jnp.int32, (BR, 128), 1)
    rloc = rowi - (rowi // ROWS) * ROWS
    flatw = rloc * 128 + lanei
    valid = flatw < n_real

    cls = jnp.where(
        s0 == mx, 0,
        jnp.where(s1 == mx, 1, jnp.where(s2 == mx, 2, 3)),
    ).astype(jnp.int32)

    score0 = jnp.where((mx >= _CONF_T) & valid, mx, -1.0)

    cmask = [cls == c for c in range(C)]
    l128v = jax.lax.broadcasted_iota(jnp.int32, (1, 128), 1)

    def pick(ref, r, li):
        row = ref[pl.ds(r, 1), :]
        return jnp.max(jnp.where(l128v == li, row, _NEG))

    rows32 = jax.lax.broadcasted_iota(jnp.int32, (B * C, 1), 0)
    lane128 = jax.lax.broadcasted_iota(jnp.int32, (B * C, 128), 1)

    def sel_rows(vals):
        return jnp.concatenate(
            [jnp.broadcast_to(v, (1, 1)) for v in vals], axis=0)

    def bsl(arr, b):
        return arr[b * ROWS:(b + 1) * ROWS, :]

    def body(t, carry):
        (score, done, sel_s, sel_v, sx1, sy1, sx2, sy2) = carry

        ms, oks, fis, bxs = [], [], [], []
        for b in range(B):
            sb = bsl(score, b)
            fb = bsl(flatw, b)
            for c in range(C):
                i = b * C + c
                cm = bsl(cmask[c], b)
                mc = jnp.max(jnp.where(cm, sb, -1.0))
                okc = jnp.logical_and(done[i] < 0.5, mc >= _CONF_T)
                eq = jnp.logical_and(sb == mc, cm)
                fic = jnp.min(jnp.where(eq, fb, _BIG))
                fic = jnp.where(okc, fic, 0)
                r = b * ROWS + fic // 128
                li = fic % 128
                ms.append(mc)
                oks.append(okc)
                fis.append(fic)
                bxs.append((pick(x1_ref, r, li), pick(y1_ref, r, li),
                            pick(x2_ref, r, li), pick(y2_ref, r, li)))

        okf = [jnp.where(o, 1.0, 0.0) for o in oks]
        a_c = [(bx[2] - bx[0]) * (bx[3] - bx[1]) for bx in bxs]

        def chain_b(vals, b):
            v = [vals[b * C + c] for c in range(C)]
            return jnp.where(
                bsl(cmask[0], b), v[0],
                jnp.where(bsl(cmask[1], b), v[1],
                          jnp.where(bsl(cmask[2], b), v[2], v[3])))

        def chain(vals):
            return jnp.concatenate([chain_b(vals, b) for b in range(B)], axis=0)

        okany = chain(okf) > 0.5
        bx1a = chain([bx[0] for bx in bxs])
        by1a = chain([bx[1] for bx in bxs])
        bx2a = chain([bx[2] for bx in bxs])
        by2a = chain([bx[3] for bx in bxs])
        asel = chain(a_c)
        fia = chain(fis)

        ix1 = jnp.maximum(bx1a, x1)
        iy1 = jnp.maximum(by1a, y1)
        ix2 = jnp.minimum(bx2a, x2)
        iy2 = jnp.minimum(by2a, y2)
        inter = jnp.maximum(ix2 - ix1, 0.0) * jnp.maximum(iy2 - iy1, 0.0)
        union = asel + area - inter
        iou = jnp.where(union > 0.0, inter / jnp.maximum(union, 1e-12), 0.0)

        decayed = score * jnp.exp(-(iou * iou) / _SIGMA)
        keep = jnp.logical_and(flatw != fia, decayed >= _CONF_T)
        new_score = jnp.where(okany, jnp.where(keep, decayed, -1.0), score)

        nd = [jnp.where(o, jnp.float32(0.0), jnp.float32(1.0)) for o in oks]

        m_v = sel_rows(ms)
        ok_v = sel_rows(okf) > 0.5
        x1_v = sel_rows([bx[0] for bx in bxs])
        y1_v = sel_rows([bx[1] for bx in bxs])
        x2_v = sel_rows([bx[2] for bx in bxs])
        y2_v = sel_rows([bx[3] for bx in bxs])

        lt = lane128 == t
        okl = jnp.logical_and(lt, ok_v)
        sel_s = jnp.where(okl, m_v, sel_s)
        sel_v = jnp.where(lt, jnp.where(ok_v, 1.0, 0.0), sel_v)
        sx1 = jnp.where(okl, x1_v, sx1)
        sy1 = jnp.where(okl, y1_v, sy1)
        sx2 = jnp.where(okl, x2_v, sx2)
        sy2 = jnp.where(okl, y2_v, sy2)
        return (new_score, nd, sel_s, sel_v, sx1, sy1, sx2, sy2)

    z = jnp.float32(0.0)
    init = (
        score0,
        [z] * (B * C),
        jnp.zeros((B * C, 128), dtype=jnp.float32),
        jnp.zeros((B * C, 128), dtype=jnp.float32),
        jnp.zeros((B * C, 128), dtype=jnp.float32),
        jnp.zeros((B * C, 128), dtype=jnp.float32),
        jnp.zeros((B * C, 128), dtype=jnp.float32),
        jnp.zeros((B * C, 128), dtype=jnp.float32),
    )
    out = jax.lax.fori_loop(0, T, body, init)
    sel_s, sel_v, sx1, sy1, sx2, sy2 = out[2:]

    cid4 = jax.lax.broadcasted_iota(jnp.int32, (C, 128), 0)
    g = cid4 * 128 + jax.lax.broadcasted_iota(jnp.int32, (C, 128), 1)
    g_f = g.astype(jnp.float32)
    l128 = jax.lax.broadcasted_iota(jnp.int32, (1, 128), 1)

    for b in range(B):
        bs = slice(b * C, (b + 1) * C)
        sel_s_b = sel_s[bs, :]
        sel_v_b = sel_v[bs, :]
        sx1_b = sx1[bs, :]
        sy1_b = sy1[bs, :]
        sx2_b = sx2[bs, :]
        sy2_b = sy2[bs, :]
        nvalid = jnp.sum(sel_v_b)
        case_b = nvalid > float(_MAX_DET)
        primary = jnp.where(case_b, sel_s_b, -g_f)

        def mbody(j, carry, primary=primary, sel_s_b=sel_s_b, sx1_b=sx1_b,
                  sy1_b=sy1_b, sx2_b=sx2_b, sy2_b=sy2_b):
            R, ox1, oy1, ox2, oy2, osc, ocl = carry
            R_b = R > 0.5
            pm = jnp.where(R_b, primary, _NEG)
            m2 = jnp.max(pm)
            any_rem = m2 > (_NEG * 0.5)
            cand = jnp.logical_and(R_b, pm == m2)
            g_sel = jnp.min(jnp.where(cand, g, 1 << 30))
            oh2 = g == g_sel
            vx1 = jnp.max(jnp.where(oh2, sx1_b, _NEG))
            vy1 = jnp.max(jnp.where(oh2, sy1_b, _NEG))
            vx2 = jnp.max(jnp.where(oh2, sx2_b, _NEG))
            vy2 = jnp.max(jnp.where(oh2, sy2_b, _NEG))
            vsc = jnp.max(jnp.where(oh2, sel_s_b, _NEG))
            vcl = jnp.max(jnp.where(oh2, cid4, -1))
            new_R = jnp.where(jnp.logical_and(oh2, any_rem), 0.0, R)
            ohj = jnp.logical_and(l128 == j, any_rem)
            ox1 = jnp.where(ohj, vx1, ox1)
            oy1 = jnp.where(ohj, vy1, oy1)
            ox2 = jnp.where(ohj, vx2, ox2)
            oy2 = jnp.where(ohj, vy2, oy2)
            osc = jnp.where(ohj, vsc, osc)
            ocl = jnp.where(ohj, vcl, ocl)
            return (new_R, ox1, oy1, ox2, oy2, osc, ocl)

        minit = (
            sel_v_b,
            jnp.zeros((1, 128), dtype=jnp.float32),
            jnp.zeros((1, 128), dtype=jnp.float32),
            jnp.zeros((1, 128), dtype=jnp.float32),
            jnp.zeros((1, 128), dtype=jnp.float32),
            jnp.zeros((1, 128), dtype=jnp.float32),
            jnp.full((1, 128), -1, dtype=jnp.int32),
        )
        (_, ox1, oy1, ox2, oy2, osc, ocl) = jax.lax.fori_loop(
            0, _MAX_DET, mbody, minit)

        zf = jnp.zeros((3, 128), dtype=jnp.float32)
        outf_ref[b] = jnp.concatenate([ox1, oy1, ox2, oy2, osc, zf], axis=0)
        zi = jnp.zeros((7, 128), dtype=jnp.int32)
        outc_ref[b] = jnp.concatenate([ocl, zi], axis=0)


def kernel(predictions, anchor_boxes):
    B, n, _ = predictions.shape
    npad = ((n + 1023) // 1024) * 1024
    ROWS = npad // 128
    BR = B * ROWS

    predT = jnp.transpose(predictions, (2, 0, 1))
    predT = jnp.pad(predT, ((0, 0), (0, 0), (0, npad - n)))
    predR = predT.reshape(8, BR, 128).reshape(8 * BR, 128)
    anchT = jnp.pad(anchor_boxes.T, ((0, 0), (0, npad - n)))
    anchR = jnp.tile(anchT[:, None, :], (1, B, 1)).reshape(4 * BR, 128)

    kfn = functools.partial(_nms_kernel, B=B, n_real=n, ROWS=ROWS)
    outf, outc = pl.pallas_call(
        kfn,
        grid=(1,),
        in_specs=[
            pl.BlockSpec((8 * BR, 128), lambda i: (0, 0)),
            pl.BlockSpec((4 * BR, 128), lambda i: (0, 0)),
        ],
        out_specs=[
            pl.BlockSpec((B, 8, 128), lambda i: (0, 0, 0)),
            pl.BlockSpec((B, 8, 128), lambda i: (0, 0, 0)),
        ],
        out_shape=[
            jax.ShapeDtypeStruct((B, 8, 128), jnp.float32),
            jax.ShapeDtypeStruct((B, 8, 128), jnp.int32),
        ],
        scratch_shapes=[
            pltpu.VMEM((BR, 128), jnp.float32),
            pltpu.VMEM((BR, 128), jnp.float32),
            pltpu.VMEM((BR, 128), jnp.float32),
            pltpu.VMEM((BR, 128), jnp.float32),
        ],
        compiler_params=pltpu.CompilerParams(
            dimension_semantics=("arbitrary",),
        ),
    )(predR, anchR)

    M = _MAX_DET
    boxes = jnp.stack(
        [outf[:, 0, :M], outf[:, 1, :M], outf[:, 2, :M], outf[:, 3, :M]], axis=-1
    )
    scores = outf[:, 4, :M]
    classes = outc[:, 0, :M]
    valid = jnp.sum((classes > -1).astype(jnp.int32), axis=1)
    idt = jax.dtypes.canonicalize_dtype(np.int64)
    return (
        valid.astype(jnp.int32),
        boxes.astype(jnp.float32),
        scores.astype(jnp.float32),
        classes.astype(idt),
    )
